# baseline (speedup 1.0000x reference)
import functools

import jax
import jax.numpy as jnp
from jax import lax
from jax.experimental import pallas as pl
from jax.experimental.pallas import tpu as pltpu
from jax.experimental.pallas import tpu_sc as plsc

N = 10000
E = 320000
EP = E + N
HEADS = 8
C1 = 16
HID = 128

NROWS = 10240
RPW = NROWS // 16
RPB = NROWS // 16
B = 128
NW = 32
NBLK = 81
EP_PAD = NW * NBLK * B
PAD_IDX = N

f32 = jnp.float32
i32 = jnp.int32

_mesh = plsc.VectorSubcoreMesh(core_axis_name="c", subcore_axis_name="s")


def _scores(h, asrc, adst, rows):
    hr = h.reshape(rows, HEADS, C1)
    s = (hr * asrc[None]).sum(-1)
    d = (hr * adst[None]).sum(-1)
    z = jnp.zeros_like(s)
    return jnp.concatenate([s, z], axis=1), jnp.concatenate([d, z], axis=1)


def _prep1_body(x_ref, w_ref, asrc_ref, adst_ref, h_ref, as_ref, ad_ref):
    h = jnp.dot(x_ref[...], w_ref[...], preferred_element_type=f32)
    h_ref[...] = h
    s, d = _scores(h, asrc_ref[...], adst_ref[...], h_ref.shape[0])
    as_ref[...] = s
    ad_ref[...] = d


def _act_in(oa_ref, ob_ref, b_ref, bid):
    act = oa_ref[...] + ob_ref[...] + b_ref[...]
    act = jnp.where(act > 0, act, jnp.exp(act) - 1.0)
    rows = bid * RPB + lax.broadcasted_iota(i32, act.shape, 0)
    return jnp.where(rows < N, act, 0.0)


def _prep2_body(oa_ref, ob_ref, b_ref, w_ref, asrc_ref, adst_ref,
                h_ref, as_ref, ad_ref):
    act = _act_in(oa_ref, ob_ref, b_ref, pl.program_id(0))
    h = jnp.dot(act, w_ref[...], preferred_element_type=f32)
    h_ref[...] = h
    s, d = _scores(h, asrc_ref[...], adst_ref[...], h_ref.shape[0])
    as_ref[...] = s
    ad_ref[...] = d


def _prep3_body(oa_ref, ob_ref, b_ref, w_ref, asrc_ref, adst_ref,
                h_ref, as_ref, ad_ref):
    act = _act_in(oa_ref, ob_ref, b_ref, pl.program_id(0))
    h = jnp.dot(act, w_ref[...], preferred_element_type=f32)
    h_ref[...] = h
    s = (h * asrc_ref[...]).sum(-1)
    d = (h * adst_ref[...]).sum(-1)
    as_ref[...] = jnp.broadcast_to(s[:, None], (RPB, 16))
    ad_ref[...] = jnp.broadcast_to(d[:, None], (RPB, 16))


def _comb_body(oa_ref, ob_ref, b_ref, o_ref):
    o_ref[...] = oa_ref[...] + ob_ref[...] + b_ref[...]


_prep1 = pl.pallas_call(
    _prep1_body,
    grid=(NROWS // RPB,),
    in_specs=[pl.BlockSpec((RPB, HID), lambda i: (i, 0)),
              pl.BlockSpec((HID, HID), lambda i: (0, 0)),
              pl.BlockSpec((HEADS, C1), lambda i: (0, 0)),
              pl.BlockSpec((HEADS, C1), lambda i: (0, 0))],
    out_specs=[pl.BlockSpec((RPB, HID), lambda i: (i, 0)),
               pl.BlockSpec((RPB, 16), lambda i: (i, 0)),
               pl.BlockSpec((RPB, 16), lambda i: (i, 0))],
    out_shape=[jax.ShapeDtypeStruct((NROWS, HID), f32),
               jax.ShapeDtypeStruct((NROWS, 16), f32),
               jax.ShapeDtypeStruct((NROWS, 16), f32)],
)

_prep2 = pl.pallas_call(
    _prep2_body,
    grid=(NROWS // RPB,),
    in_specs=[pl.BlockSpec((RPB, HID), lambda i: (i, 0)),
              pl.BlockSpec((RPB, HID), lambda i: (i, 0)),
              pl.BlockSpec((1, HID), lambda i: (0, 0)),
              pl.BlockSpec((HID, HID), lambda i: (0, 0)),
              pl.BlockSpec((HEADS, C1), lambda i: (0, 0)),
              pl.BlockSpec((HEADS, C1), lambda i: (0, 0))],
    out_specs=[pl.BlockSpec((RPB, HID), lambda i: (i, 0)),
               pl.BlockSpec((RPB, 16), lambda i: (i, 0)),
               pl.BlockSpec((RPB, 16), lambda i: (i, 0))],
    out_shape=[jax.ShapeDtypeStruct((NROWS, HID), f32),
               jax.ShapeDtypeStruct((NROWS, 16), f32),
               jax.ShapeDtypeStruct((NROWS, 16), f32)],
)

_prep3 = pl.pallas_call(
    _prep3_body,
    grid=(NROWS // RPB,),
    in_specs=[pl.BlockSpec((RPB, HID), lambda i: (i, 0)),
              pl.BlockSpec((RPB, HID), lambda i: (i, 0)),
              pl.BlockSpec((1, HID), lambda i: (0, 0)),
              pl.BlockSpec((HID, 16), lambda i: (0, 0)),
              pl.BlockSpec((1, 16), lambda i: (0, 0)),
              pl.BlockSpec((1, 16), lambda i: (0, 0))],
    out_specs=[pl.BlockSpec((RPB, 16), lambda i: (i, 0)),
               pl.BlockSpec((RPB, 16), lambda i: (i, 0)),
               pl.BlockSpec((RPB, 16), lambda i: (i, 0))],
    out_shape=[jax.ShapeDtypeStruct((NROWS, 16), f32),
               jax.ShapeDtypeStruct((NROWS, 16), f32),
               jax.ShapeDtypeStruct((NROWS, 16), f32)],
)

_combine = pl.pallas_call(
    _comb_body,
    out_shape=jax.ShapeDtypeStruct((NROWS, 16), f32),
)


def _p1_body(src_hbm, dst_hbm, as_hbm, ad_hbm, z16_hbm,
             p_hbm, dena_hbm, denb_hbm,
             sidx, didx, as_v, ad_v, p_v, den_sh,
             sem_idx, sem_in):
    c = lax.axis_index("c")
    s = lax.axis_index("s")
    wid = c * 16 + s
    pltpu.sync_copy(z16_hbm.at[pl.ds(s * RPW, RPW)],
                    den_sh.at[pl.ds(s * RPW, RPW)])
    plsc.subcore_barrier()
    base_w = wid * (NBLK * B)

    def fire_idx(j, d4):
        base = base_w + j * B
        pltpu.async_copy(src_hbm.at[pl.ds(base, B)], sidx.at[d4],
                         sem_idx.at[d4])
        pltpu.async_copy(dst_hbm.at[pl.ds(base, B)], didx.at[d4],
                         sem_idx.at[d4])

    def wait_idx(j, d4):
        base = base_w + j * B
        pltpu.make_async_copy(src_hbm.at[pl.ds(base, B)], sidx.at[d4],
                              sem_idx.at[d4]).wait()
        pltpu.make_async_copy(dst_hbm.at[pl.ds(base, B)], didx.at[d4],
                              sem_idx.at[d4]).wait()

    def fire_gather(d4, d):
        pltpu.async_copy(as_hbm.at[sidx.at[d4]], as_v.at[d], sem_in.at[d])
        pltpu.async_copy(ad_hbm.at[didx.at[d4]], ad_v.at[d], sem_in.at[d])

    def wait_gather(d4, d):
        pltpu.make_async_copy(as_hbm.at[sidx.at[d4]], as_v.at[d],
                              sem_in.at[d]).wait()
        pltpu.make_async_copy(ad_hbm.at[didx.at[d4]], ad_v.at[d],
                              sem_in.at[d]).wait()

    def do_block(j, k):
        d4, d = k, k % 2
        base = base_w + j * B
        wait_gather(d4, d)

        @pl.when(j + 1 < NBLK)
        def _():
            wait_idx(j + 1, (k + 1) % 4)
            fire_gather((k + 1) % 4, (k + 1) % 2)

        @plsc.parallel_loop(0, B, unroll=4)
        def _(b):
            e = as_v[d, b, :] + ad_v[d, b, :]
            e = jnp.maximum(e, 0.2 * e)
            p_v[d, b, :] = jnp.exp(e)

        pltpu.sync_copy(p_v.at[d], p_hbm.at[pl.ds(base, B)])
        pltpu.sync_copy(p_v.at[d], den_sh.at[didx.at[d4]], add=True)

        @pl.when(j + 2 < NBLK)
        def _():
            fire_idx(j + 2, (k + 2) % 4)

    fire_idx(0, 0)
    fire_idx(1, 1)
    wait_idx(0, 0)
    fire_gather(0, 0)

    def quad(q, carry):
        j0 = 4 * q
        for k in range(4):
            do_block(j0 + k, k)
        return carry
    lax.fori_loop(0, NBLK // 4, quad, 0)
    do_block(NBLK - 1, 0)
    plsc.subcore_barrier()

    @pl.when(c == 0)
    def _():
        pltpu.sync_copy(den_sh.at[pl.ds(s * RPW, RPW)],
                        dena_hbm.at[pl.ds(s * RPW, RPW)])

    @pl.when(c == 1)
    def _():
        pltpu.sync_copy(den_sh.at[pl.ds(s * RPW, RPW)],
                        denb_hbm.at[pl.ds(s * RPW, RPW)])


_pass1 = pl.kernel(
    _p1_body,
    out_type=(jax.ShapeDtypeStruct((EP_PAD, 16), f32),
              jax.ShapeDtypeStruct((NROWS, 16), f32),
              jax.ShapeDtypeStruct((NROWS, 16), f32)),
    mesh=_mesh,
    scratch_types=[pltpu.VMEM((4, B), i32), pltpu.VMEM((4, B), i32),
                   pltpu.VMEM((2, B, 16), f32), pltpu.VMEM((2, B, 16), f32),
                   pltpu.VMEM((2, B, 16), f32),
                   pltpu.VMEM_SHARED((NROWS, 16), f32),
                   pltpu.SemaphoreType.DMA((4,)),
                   pltpu.SemaphoreType.DMA((2,))],
    compiler_params=pltpu.CompilerParams(use_tc_tiling_on_sc=False),
)


def _p2_body(src_hbm, dst_hbm, p_hbm, rd_hbm, h_hbm, z_hbm,
             alpha_hbm, oa_hbm, ob_hbm,
             sidx, didx, p_v, d0_v, pk_v, rows_v, out_sh,
             sem_idx, sem_in,
             wide):
    c = lax.axis_index("c")
    s = lax.axis_index("s")
    wid = c * 16 + s
    pltpu.sync_copy(z_hbm.at[pl.ds(s * RPW, RPW)],
                    out_sh.at[pl.ds(s * RPW, RPW)])
    plsc.subcore_barrier()
    base_w = wid * (NBLK * B)

    def fire_idx(j, d4):
        base = base_w + j * B
        pltpu.async_copy(src_hbm.at[pl.ds(base, B)], sidx.at[d4],
                         sem_idx.at[d4])
        pltpu.async_copy(dst_hbm.at[pl.ds(base, B)], didx.at[d4],
                         sem_idx.at[d4])

    def wait_idx(j, d4):
        base = base_w + j * B
        pltpu.make_async_copy(src_hbm.at[pl.ds(base, B)], sidx.at[d4],
                              sem_idx.at[d4]).wait()
        pltpu.make_async_copy(dst_hbm.at[pl.ds(base, B)], didx.at[d4],
                              sem_idx.at[d4]).wait()

    def fire_gather(j, d4, d):
        base = base_w + j * B
        pltpu.async_copy(h_hbm.at[sidx.at[d4]], rows_v.at[d], sem_in.at[d])
        pltpu.async_copy(rd_hbm.at[didx.at[d4]], d0_v.at[d], sem_in.at[d])
        pltpu.async_copy(p_hbm.at[pl.ds(base, B)], p_v.at[d], sem_in.at[d])

    def wait_gather(j, d4, d):
        base = base_w + j * B
        pltpu.make_async_copy(h_hbm.at[sidx.at[d4]], rows_v.at[d],
                              sem_in.at[d]).wait()
        pltpu.make_async_copy(rd_hbm.at[didx.at[d4]], d0_v.at[d],
                              sem_in.at[d]).wait()
        pltpu.make_async_copy(p_hbm.at[pl.ds(base, B)], p_v.at[d],
                              sem_in.at[d]).wait()

    lane = lax.iota(i32, 16)
    low8 = lane & 7

    def do_block(j, k):
        d4, d = k, k % 2
        base = base_w + j * B
        wait_gather(j, d4, d)

        @pl.when(j + 1 < NBLK)
        def _():
            wait_idx(j + 1, (k + 1) % 4)
            fire_gather(j + 1, (k + 1) % 4, (k + 1) % 2)

        @plsc.parallel_loop(0, B // 2, unroll=2)
        def _(i):
            b0 = 2 * i
            b1 = 2 * i + 1
            a0 = p_v[d, b0, :] * d0_v[d, b0, :]
            a1 = p_v[d, b1, :] * d0_v[d, b1, :]
            a1lo = a1.at[low8].get(mode="promise_in_bounds")
            pk_v[d, i, :] = jnp.where(lane < 8, a0, a1lo)
            if wide:
                for hh in range(HEADS):
                    sl = pl.ds(hh * 16, 16)
                    rows_v[d, b0, sl] = rows_v[d, b0, sl] * a0[hh]
                    rows_v[d, b1, sl] = rows_v[d, b1, sl] * a1[hh]
            else:
                rows_v[d, b0, :] = rows_v[d, b0, :] * a0
                rows_v[d, b1, :] = rows_v[d, b1, :] * a1

        pltpu.sync_copy(pk_v.at[d], alpha_hbm.at[pl.ds(base // 2, B // 2)])
        pltpu.sync_copy(rows_v.at[d], out_sh.at[didx.at[d4]], add=True)

        @pl.when(j + 2 < NBLK)
        def _():
            fire_idx(j + 2, (k + 2) % 4)

    fire_idx(0, 0)
    fire_idx(1, 1)
    wait_idx(0, 0)
    fire_gather(0, 0, 0)

    def quad(q, carry):
        j0 = 4 * q
        for k in range(4):
            do_block(j0 + k, k)
        return carry
    lax.fori_loop(0, NBLK // 4, quad, 0)
    do_block(NBLK - 1, 0)
    plsc.subcore_barrier()

    @pl.when(c == 0)
    def _():
        pltpu.sync_copy(out_sh.at[pl.ds(s * RPW, RPW)],
                        oa_hbm.at[pl.ds(s * RPW, RPW)])

    @pl.when(c == 1)
    def _():
        pltpu.sync_copy(out_sh.at[pl.ds(s * RPW, RPW)],
                        ob_hbm.at[pl.ds(s * RPW, RPW)])


def _make_pass2(w):
    return pl.kernel(
        functools.partial(_p2_body, wide=(w == HID)),
        out_type=(jax.ShapeDtypeStruct((EP_PAD // 2, 16), f32),
                  jax.ShapeDtypeStruct((NROWS, w), f32),
                  jax.ShapeDtypeStruct((NROWS, w), f32)),
        mesh=_mesh,
        scratch_types=[pltpu.VMEM((4, B), i32), pltpu.VMEM((4, B), i32),
                       pltpu.VMEM((2, B, 16), f32), pltpu.VMEM((2, B, 16), f32),
                       pltpu.VMEM((2, B // 2, 16), f32),
                       pltpu.VMEM((2, B, w), f32),
                       pltpu.VMEM_SHARED((NROWS, w), f32),
                       pltpu.SemaphoreType.DMA((4,)),
                       pltpu.SemaphoreType.DMA((2,))],
        compiler_params=pltpu.CompilerParams(use_tc_tiling_on_sc=False),
    )


_pass2_big = _make_pass2(HID)
_pass2_small = _make_pass2(16)


def kernel(x, edge_index, W1, as1, ad1, b1, W2, as2, ad2, b2, W3, as3, ad3, b3):
    loops = jnp.arange(N, dtype=i32)
    pad = jnp.full((EP_PAD - EP,), PAD_IDX, i32)
    src = jnp.concatenate([edge_index[0].astype(i32), loops, pad])
    dst = jnp.concatenate([edge_index[1].astype(i32), loops, pad])
    xp = jnp.pad(x, ((0, NROWS - N), (0, 0)))
    z16 = jnp.zeros((NROWS, 16), f32)
    z128 = jnp.zeros((NROWS, HID), f32)

    h1, as1t, ad1t = _prep1(xp, W1, as1, ad1)
    p1, d1a, d1b = _pass1(src, dst, as1t, ad1t, z16)
    rd1 = 1.0 / (d1a + d1b + 1e-16)
    a1f, o1a, o1b = _pass2_big(src, dst, p1, rd1, h1, z128)

    h2, as2t, ad2t = _prep2(o1a, o1b, b1.reshape(1, HID), W2, as2, ad2)
    p2, d2a, d2b = _pass1(src, dst, as2t, ad2t, z16)
    rd2 = 1.0 / (d2a + d2b + 1e-16)
    a2f, o2a, o2b = _pass2_big(src, dst, p2, rd2, h2, z128)

    W3p = jnp.pad(W3, ((0, 0), (0, 14)))
    as3p = jnp.pad(as3, ((0, 0), (0, 14)))
    ad3p = jnp.pad(ad3, ((0, 0), (0, 14)))
    h3t, as3t, ad3t = _prep3(o2a, o2b, b2.reshape(1, HID), W3p, as3p, ad3p)
    p3, d3a, d3b = _pass1(src, dst, as3t, ad3t, z16)
    rd3 = 1.0 / (d3a + d3b + 1e-16)
    a3f, o3a, o3b = _pass2_small(src, dst, p3, rd3, h3t, z16)

    out_full = _combine(o3a, o3b, jnp.pad(b3, (0, 14)).reshape(1, 16))
    a1 = a1f.reshape(EP_PAD, HEADS)[:EP]
    a2 = a2f.reshape(EP_PAD, HEADS)[:EP]
    a3 = a3f.reshape(EP_PAD, HEADS)[:EP, :1]
    return (out_full[:N, :2], a1, a2, a3)


# speedup vs baseline: 1.2219x; 1.2219x over previous
import functools

import jax
import jax.numpy as jnp
from jax import lax
from jax.experimental import pallas as pl
from jax.experimental.pallas import tpu as pltpu
from jax.experimental.pallas import tpu_sc as plsc

N = 10000
E = 320000
EP = E + N
HEADS = 8
C1 = 16
HID = 128

NROWS = 10240
RPW = NROWS // 16
RPB = NROWS // 16
B = 128
NW = 32
NBLK = 81
EP_PAD = NW * NBLK * B
PAD_IDX = N

f32 = jnp.float32
i32 = jnp.int32

_mesh = plsc.VectorSubcoreMesh(core_axis_name="c", subcore_axis_name="s")


def _scores(h, asrc, adst, rows):
    hr = h.reshape(rows, HEADS, C1)
    s = (hr * asrc[None]).sum(-1)
    d = (hr * adst[None]).sum(-1)
    z = jnp.zeros_like(s)
    return jnp.concatenate([s, z], axis=1), jnp.concatenate([d, z], axis=1)


def _prep1_body(x_ref, w_ref, asrc_ref, adst_ref, h_ref, as_ref, ad_ref):
    h = jnp.dot(x_ref[...], w_ref[...], preferred_element_type=f32)
    h_ref[...] = h
    s, d = _scores(h, asrc_ref[...], adst_ref[...], h_ref.shape[0])
    as_ref[...] = s
    ad_ref[...] = d


def _act_in(oa_ref, ob_ref, b_ref, bid):
    act = oa_ref[...] + ob_ref[...] + b_ref[...]
    act = jnp.where(act > 0, act, jnp.exp(act) - 1.0)
    rows = bid * RPB + lax.broadcasted_iota(i32, act.shape, 0)
    return jnp.where(rows < N, act, 0.0)


def _prep2_body(oa_ref, ob_ref, b_ref, w_ref, asrc_ref, adst_ref,
                h_ref, as_ref, ad_ref):
    act = _act_in(oa_ref, ob_ref, b_ref, pl.program_id(0))
    h = jnp.dot(act, w_ref[...], preferred_element_type=f32)
    h_ref[...] = h
    s, d = _scores(h, asrc_ref[...], adst_ref[...], h_ref.shape[0])
    as_ref[...] = s
    ad_ref[...] = d


def _prep3_body(oa_ref, ob_ref, b_ref, w_ref, asrc_ref, adst_ref,
                h_ref, as_ref, ad_ref):
    act = _act_in(oa_ref, ob_ref, b_ref, pl.program_id(0))
    h = jnp.dot(act, w_ref[...], preferred_element_type=f32)
    h_ref[...] = h
    s = (h * asrc_ref[...]).sum(-1)
    d = (h * adst_ref[...]).sum(-1)
    as_ref[...] = jnp.broadcast_to(s[:, None], (RPB, 16))
    ad_ref[...] = jnp.broadcast_to(d[:, None], (RPB, 16))


def _comb_body(oa_ref, ob_ref, b_ref, o_ref):
    o_ref[...] = oa_ref[...] + ob_ref[...] + b_ref[...]


_prep1 = pl.pallas_call(
    _prep1_body,
    grid=(NROWS // RPB,),
    in_specs=[pl.BlockSpec((RPB, HID), lambda i: (i, 0)),
              pl.BlockSpec((HID, HID), lambda i: (0, 0)),
              pl.BlockSpec((HEADS, C1), lambda i: (0, 0)),
              pl.BlockSpec((HEADS, C1), lambda i: (0, 0))],
    out_specs=[pl.BlockSpec((RPB, HID), lambda i: (i, 0)),
               pl.BlockSpec((RPB, 16), lambda i: (i, 0)),
               pl.BlockSpec((RPB, 16), lambda i: (i, 0))],
    out_shape=[jax.ShapeDtypeStruct((NROWS, HID), f32),
               jax.ShapeDtypeStruct((NROWS, 16), f32),
               jax.ShapeDtypeStruct((NROWS, 16), f32)],
)

_prep2 = pl.pallas_call(
    _prep2_body,
    grid=(NROWS // RPB,),
    in_specs=[pl.BlockSpec((RPB, HID), lambda i: (i, 0)),
              pl.BlockSpec((RPB, HID), lambda i: (i, 0)),
              pl.BlockSpec((1, HID), lambda i: (0, 0)),
              pl.BlockSpec((HID, HID), lambda i: (0, 0)),
              pl.BlockSpec((HEADS, C1), lambda i: (0, 0)),
              pl.BlockSpec((HEADS, C1), lambda i: (0, 0))],
    out_specs=[pl.BlockSpec((RPB, HID), lambda i: (i, 0)),
               pl.BlockSpec((RPB, 16), lambda i: (i, 0)),
               pl.BlockSpec((RPB, 16), lambda i: (i, 0))],
    out_shape=[jax.ShapeDtypeStruct((NROWS, HID), f32),
               jax.ShapeDtypeStruct((NROWS, 16), f32),
               jax.ShapeDtypeStruct((NROWS, 16), f32)],
)

_prep3 = pl.pallas_call(
    _prep3_body,
    grid=(NROWS // RPB,),
    in_specs=[pl.BlockSpec((RPB, HID), lambda i: (i, 0)),
              pl.BlockSpec((RPB, HID), lambda i: (i, 0)),
              pl.BlockSpec((1, HID), lambda i: (0, 0)),
              pl.BlockSpec((HID, 16), lambda i: (0, 0)),
              pl.BlockSpec((1, 16), lambda i: (0, 0)),
              pl.BlockSpec((1, 16), lambda i: (0, 0))],
    out_specs=[pl.BlockSpec((RPB, 16), lambda i: (i, 0)),
               pl.BlockSpec((RPB, 16), lambda i: (i, 0)),
               pl.BlockSpec((RPB, 16), lambda i: (i, 0))],
    out_shape=[jax.ShapeDtypeStruct((NROWS, 16), f32),
               jax.ShapeDtypeStruct((NROWS, 16), f32),
               jax.ShapeDtypeStruct((NROWS, 16), f32)],
)

_combine = pl.pallas_call(
    _comb_body,
    out_shape=jax.ShapeDtypeStruct((NROWS, 16), f32),
)


def _p1_body(src_hbm, dst_hbm, as_hbm, ad_hbm, z16_hbm,
             p_hbm, dena_hbm, denb_hbm,
             sidx, didx, as_v, ad_v, p_v, den_sh,
             sem_idx, sem_in):
    c = lax.axis_index("c")
    s = lax.axis_index("s")
    wid = c * 16 + s
    pltpu.sync_copy(z16_hbm.at[pl.ds(s * RPW, RPW)],
                    den_sh.at[pl.ds(s * RPW, RPW)])
    plsc.subcore_barrier()
    base_w = wid * (NBLK * B)

    def fire_idx(j, d4):
        base = base_w + j * B
        pltpu.async_copy(src_hbm.at[pl.ds(base, B)], sidx.at[d4],
                         sem_idx.at[d4])
        pltpu.async_copy(dst_hbm.at[pl.ds(base, B)], didx.at[d4],
                         sem_idx.at[d4])

    def wait_idx(j, d4):
        base = base_w + j * B
        pltpu.make_async_copy(src_hbm.at[pl.ds(base, B)], sidx.at[d4],
                              sem_idx.at[d4]).wait()
        pltpu.make_async_copy(dst_hbm.at[pl.ds(base, B)], didx.at[d4],
                              sem_idx.at[d4]).wait()

    def fire_gather(d4, d):
        pltpu.async_copy(as_hbm.at[sidx.at[d4]], as_v.at[d], sem_in.at[d])
        pltpu.async_copy(ad_hbm.at[didx.at[d4]], ad_v.at[d], sem_in.at[d])

    def wait_gather(d4, d):
        pltpu.make_async_copy(as_hbm.at[sidx.at[d4]], as_v.at[d],
                              sem_in.at[d]).wait()
        pltpu.make_async_copy(ad_hbm.at[didx.at[d4]], ad_v.at[d],
                              sem_in.at[d]).wait()

    def do_block(j, k):
        d4, d = k, k % 2
        base = base_w + j * B
        wait_gather(d4, d)

        @pl.when(j + 1 < NBLK)
        def _():
            wait_idx(j + 1, (k + 1) % 4)
            fire_gather((k + 1) % 4, (k + 1) % 2)

        @plsc.parallel_loop(0, B, unroll=4)
        def _(b):
            e = as_v[d, b, :] + ad_v[d, b, :]
            e = jnp.maximum(e, 0.2 * e)
            p_v[d, b, :] = jnp.exp(e)

        pltpu.sync_copy(p_v.at[d], p_hbm.at[pl.ds(base, B)])
        pltpu.sync_copy(p_v.at[d], den_sh.at[didx.at[d4]], add=True)

        @pl.when(j + 2 < NBLK)
        def _():
            fire_idx(j + 2, (k + 2) % 4)

    fire_idx(0, 0)
    fire_idx(1, 1)
    wait_idx(0, 0)
    fire_gather(0, 0)

    def quad(q, carry):
        j0 = 4 * q
        for k in range(4):
            do_block(j0 + k, k)
        return carry
    lax.fori_loop(0, NBLK // 4, quad, 0)
    do_block(NBLK - 1, 0)
    plsc.subcore_barrier()

    @pl.when(c == 0)
    def _():
        pltpu.sync_copy(den_sh.at[pl.ds(s * RPW, RPW)],
                        dena_hbm.at[pl.ds(s * RPW, RPW)])

    @pl.when(c == 1)
    def _():
        pltpu.sync_copy(den_sh.at[pl.ds(s * RPW, RPW)],
                        denb_hbm.at[pl.ds(s * RPW, RPW)])


_pass1 = pl.kernel(
    _p1_body,
    out_type=(jax.ShapeDtypeStruct((EP_PAD, 16), f32),
              jax.ShapeDtypeStruct((NROWS, 16), f32),
              jax.ShapeDtypeStruct((NROWS, 16), f32)),
    mesh=_mesh,
    scratch_types=[pltpu.VMEM((4, B), i32), pltpu.VMEM((4, B), i32),
                   pltpu.VMEM((2, B, 16), f32), pltpu.VMEM((2, B, 16), f32),
                   pltpu.VMEM((2, B, 16), f32),
                   pltpu.VMEM_SHARED((NROWS, 16), f32),
                   pltpu.SemaphoreType.DMA((4,)),
                   pltpu.SemaphoreType.DMA((2,))],
    compiler_params=pltpu.CompilerParams(use_tc_tiling_on_sc=False),
)


def _p2_body(src_hbm, dst_hbm, p_hbm, rd_hbm, h_hbm, z_hbm,
             alpha_hbm, oa_hbm, ob_hbm,
             sidx, didx, p_v, d0_v, pk_v, rows_v, out_sh,
             sem_idx, sem_in,
             wide):
    c = lax.axis_index("c")
    s = lax.axis_index("s")
    wid = c * 16 + s
    pltpu.sync_copy(z_hbm.at[pl.ds(s * RPW, RPW)],
                    out_sh.at[pl.ds(s * RPW, RPW)])
    plsc.subcore_barrier()
    base_w = wid * (NBLK * B)

    def fire_idx(j, d4):
        base = base_w + j * B
        pltpu.async_copy(src_hbm.at[pl.ds(base, B)], sidx.at[d4],
                         sem_idx.at[d4])
        pltpu.async_copy(dst_hbm.at[pl.ds(base, B)], didx.at[d4],
                         sem_idx.at[d4])

    def wait_idx(j, d4):
        base = base_w + j * B
        pltpu.make_async_copy(src_hbm.at[pl.ds(base, B)], sidx.at[d4],
                              sem_idx.at[d4]).wait()
        pltpu.make_async_copy(dst_hbm.at[pl.ds(base, B)], didx.at[d4],
                              sem_idx.at[d4]).wait()

    def fire_gather(j, d4, d):
        base = base_w + j * B
        pltpu.async_copy(h_hbm.at[sidx.at[d4]], rows_v.at[d], sem_in.at[d])
        pltpu.async_copy(rd_hbm.at[didx.at[d4]], d0_v.at[d], sem_in.at[d])
        pltpu.async_copy(p_hbm.at[pl.ds(base, B)], p_v.at[d], sem_in.at[d])

    def wait_gather(j, d4, d):
        base = base_w + j * B
        pltpu.make_async_copy(h_hbm.at[sidx.at[d4]], rows_v.at[d],
                              sem_in.at[d]).wait()
        pltpu.make_async_copy(rd_hbm.at[didx.at[d4]], d0_v.at[d],
                              sem_in.at[d]).wait()
        pltpu.make_async_copy(p_hbm.at[pl.ds(base, B)], p_v.at[d],
                              sem_in.at[d]).wait()

    lane = lax.iota(i32, 16)
    low8 = lane & 7

    def do_block(j, k):
        d4, d = k, k % 2
        base = base_w + j * B
        wait_gather(j, d4, d)

        @pl.when(j + 1 < NBLK)
        def _():
            wait_idx(j + 1, (k + 1) % 4)
            fire_gather(j + 1, (k + 1) % 4, (k + 1) % 2)

        @plsc.parallel_loop(0, B // 2, unroll=2)
        def _(i):
            b0 = 2 * i
            b1 = 2 * i + 1
            a0 = p_v[d, b0, :] * d0_v[d, b0, :]
            a1 = p_v[d, b1, :] * d0_v[d, b1, :]
            a1lo = a1.at[low8].get(mode="promise_in_bounds")
            pk_v[d, pl.ds(i * 16, 16)] = jnp.where(lane < 8, a0, a1lo)
            if wide:
                for hh in range(HEADS):
                    sl = pl.ds(hh * 16, 16)
                    rows_v[d, b0, sl] = rows_v[d, b0, sl] * a0[hh]
                    rows_v[d, b1, sl] = rows_v[d, b1, sl] * a1[hh]
            else:
                rows_v[d, b0, :] = rows_v[d, b0, :] * a0
                rows_v[d, b1, :] = rows_v[d, b1, :] * a1

        pltpu.sync_copy(pk_v.at[d], alpha_hbm.at[pl.ds(base * 8, B * 8)])
        pltpu.sync_copy(rows_v.at[d], out_sh.at[didx.at[d4]], add=True)

        @pl.when(j + 2 < NBLK)
        def _():
            fire_idx(j + 2, (k + 2) % 4)

    fire_idx(0, 0)
    fire_idx(1, 1)
    wait_idx(0, 0)
    fire_gather(0, 0, 0)

    def quad(q, carry):
        j0 = 4 * q
        for k in range(4):
            do_block(j0 + k, k)
        return carry
    lax.fori_loop(0, NBLK // 4, quad, 0)
    do_block(NBLK - 1, 0)
    plsc.subcore_barrier()

    @pl.when(c == 0)
    def _():
        pltpu.sync_copy(out_sh.at[pl.ds(s * RPW, RPW)],
                        oa_hbm.at[pl.ds(s * RPW, RPW)])

    @pl.when(c == 1)
    def _():
        pltpu.sync_copy(out_sh.at[pl.ds(s * RPW, RPW)],
                        ob_hbm.at[pl.ds(s * RPW, RPW)])


def _make_pass2(w):
    return pl.kernel(
        functools.partial(_p2_body, wide=(w == HID)),
        out_type=(jax.ShapeDtypeStruct((EP_PAD * 8,), f32),
                  jax.ShapeDtypeStruct((NROWS, w), f32),
                  jax.ShapeDtypeStruct((NROWS, w), f32)),
        mesh=_mesh,
        scratch_types=[pltpu.VMEM((4, B), i32), pltpu.VMEM((4, B), i32),
                       pltpu.VMEM((2, B, 16), f32), pltpu.VMEM((2, B, 16), f32),
                       pltpu.VMEM((2, B * 8), f32),
                       pltpu.VMEM((2, B, w), f32),
                       pltpu.VMEM_SHARED((NROWS, w), f32),
                       pltpu.SemaphoreType.DMA((4,)),
                       pltpu.SemaphoreType.DMA((2,))],
        compiler_params=pltpu.CompilerParams(use_tc_tiling_on_sc=False),
    )


_pass2_big = _make_pass2(HID)
_pass2_small = _make_pass2(16)


def kernel(x, edge_index, W1, as1, ad1, b1, W2, as2, ad2, b2, W3, as3, ad3, b3):
    loops = jnp.arange(N, dtype=i32)
    pad = jnp.full((EP_PAD - EP,), PAD_IDX, i32)
    src = jnp.concatenate([edge_index[0].astype(i32), loops, pad])
    dst = jnp.concatenate([edge_index[1].astype(i32), loops, pad])
    xp = jnp.pad(x, ((0, NROWS - N), (0, 0)))
    z16 = jnp.zeros((NROWS, 16), f32)
    z128 = jnp.zeros((NROWS, HID), f32)

    h1, as1t, ad1t = _prep1(xp, W1, as1, ad1)
    p1, d1a, d1b = _pass1(src, dst, as1t, ad1t, z16)
    rd1 = 1.0 / (d1a + d1b + 1e-16)
    a1f, o1a, o1b = _pass2_big(src, dst, p1, rd1, h1, z128)

    h2, as2t, ad2t = _prep2(o1a, o1b, b1.reshape(1, HID), W2, as2, ad2)
    p2, d2a, d2b = _pass1(src, dst, as2t, ad2t, z16)
    rd2 = 1.0 / (d2a + d2b + 1e-16)
    a2f, o2a, o2b = _pass2_big(src, dst, p2, rd2, h2, z128)

    W3p = jnp.pad(W3, ((0, 0), (0, 14)))
    as3p = jnp.pad(as3, ((0, 0), (0, 14)))
    ad3p = jnp.pad(ad3, ((0, 0), (0, 14)))
    h3t, as3t, ad3t = _prep3(o2a, o2b, b2.reshape(1, HID), W3p, as3p, ad3p)
    p3, d3a, d3b = _pass1(src, dst, as3t, ad3t, z16)
    rd3 = 1.0 / (d3a + d3b + 1e-16)
    a3f, o3a, o3b = _pass2_small(src, dst, p3, rd3, h3t, z16)

    out_full = _combine(o3a, o3b, jnp.pad(b3, (0, 14)).reshape(1, 16))
    a1 = a1f[:EP * HEADS].reshape(EP, HEADS)
    a2 = a2f[:EP * HEADS].reshape(EP, HEADS)
    a3 = a3f[:EP * HEADS].reshape(EP, HEADS)[:, :1]
    return (out_full[:N, :2], a1, a2, a3)


# speedup vs baseline: 1.3783x; 1.1280x over previous
import functools

import jax
import jax.numpy as jnp
from jax import lax
from jax.experimental import pallas as pl
from jax.experimental.pallas import tpu as pltpu
from jax.experimental.pallas import tpu_sc as plsc

N = 10000
E = 320000
EP = E + N
HEADS = 8
C1 = 16
HID = 128

NROWS = 10240
RPW = NROWS // 16
RPB = NROWS // 16
B = 128
NW = 32
NBLK = 81
EP_PAD = NW * NBLK * B
PAD_IDX = N

f32 = jnp.float32
i32 = jnp.int32

_mesh = plsc.VectorSubcoreMesh(core_axis_name="c", subcore_axis_name="s")


def _scores(h, asrc, adst, rows):
    hr = h.reshape(rows, HEADS, C1)
    s = (hr * asrc[None]).sum(-1)
    d = (hr * adst[None]).sum(-1)
    z = jnp.zeros_like(s)
    return jnp.concatenate([s, z], axis=1), jnp.concatenate([d, z], axis=1)


def _prep1_body(x_ref, w_ref, asrc_ref, adst_ref, h_ref, as_ref, ad_ref):
    h = jnp.dot(x_ref[...], w_ref[...], preferred_element_type=f32)
    h_ref[...] = h
    s, d = _scores(h, asrc_ref[...], adst_ref[...], h_ref.shape[0])
    as_ref[...] = s
    ad_ref[...] = d


def _act_in(oa_ref, ob_ref, b_ref, bid):
    act = oa_ref[...] + ob_ref[...] + b_ref[...]
    act = jnp.where(act > 0, act, jnp.exp(act) - 1.0)
    rows = bid * RPB + lax.broadcasted_iota(i32, act.shape, 0)
    return jnp.where(rows < N, act, 0.0)


def _prep2_body(oa_ref, ob_ref, b_ref, w_ref, asrc_ref, adst_ref,
                h_ref, as_ref, ad_ref):
    act = _act_in(oa_ref, ob_ref, b_ref, pl.program_id(0))
    h = jnp.dot(act, w_ref[...], preferred_element_type=f32)
    h_ref[...] = h
    s, d = _scores(h, asrc_ref[...], adst_ref[...], h_ref.shape[0])
    as_ref[...] = s
    ad_ref[...] = d


def _prep3_body(oa_ref, ob_ref, b_ref, w_ref, asrc_ref, adst_ref,
                h_ref, as_ref, ad_ref):
    act = _act_in(oa_ref, ob_ref, b_ref, pl.program_id(0))
    h = jnp.dot(act, w_ref[...], preferred_element_type=f32)
    h_ref[...] = h
    s = (h * asrc_ref[...]).sum(-1)
    d = (h * adst_ref[...]).sum(-1)
    as_ref[...] = jnp.broadcast_to(s[:, None], (RPB, 16))
    ad_ref[...] = jnp.broadcast_to(d[:, None], (RPB, 16))


def _comb_body(oa_ref, ob_ref, b_ref, o_ref):
    o_ref[...] = oa_ref[...] + ob_ref[...] + b_ref[...]


_prep1 = pl.pallas_call(
    _prep1_body,
    grid=(NROWS // RPB,),
    in_specs=[pl.BlockSpec((RPB, HID), lambda i: (i, 0)),
              pl.BlockSpec((HID, HID), lambda i: (0, 0)),
              pl.BlockSpec((HEADS, C1), lambda i: (0, 0)),
              pl.BlockSpec((HEADS, C1), lambda i: (0, 0))],
    out_specs=[pl.BlockSpec((RPB, HID), lambda i: (i, 0)),
               pl.BlockSpec((RPB, 16), lambda i: (i, 0)),
               pl.BlockSpec((RPB, 16), lambda i: (i, 0))],
    out_shape=[jax.ShapeDtypeStruct((NROWS, HID), f32),
               jax.ShapeDtypeStruct((NROWS, 16), f32),
               jax.ShapeDtypeStruct((NROWS, 16), f32)],
)

_prep2 = pl.pallas_call(
    _prep2_body,
    grid=(NROWS // RPB,),
    in_specs=[pl.BlockSpec((RPB, HID), lambda i: (i, 0)),
              pl.BlockSpec((RPB, HID), lambda i: (i, 0)),
              pl.BlockSpec((1, HID), lambda i: (0, 0)),
              pl.BlockSpec((HID, HID), lambda i: (0, 0)),
              pl.BlockSpec((HEADS, C1), lambda i: (0, 0)),
              pl.BlockSpec((HEADS, C1), lambda i: (0, 0))],
    out_specs=[pl.BlockSpec((RPB, HID), lambda i: (i, 0)),
               pl.BlockSpec((RPB, 16), lambda i: (i, 0)),
               pl.BlockSpec((RPB, 16), lambda i: (i, 0))],
    out_shape=[jax.ShapeDtypeStruct((NROWS, HID), f32),
               jax.ShapeDtypeStruct((NROWS, 16), f32),
               jax.ShapeDtypeStruct((NROWS, 16), f32)],
)

_prep3 = pl.pallas_call(
    _prep3_body,
    grid=(NROWS // RPB,),
    in_specs=[pl.BlockSpec((RPB, HID), lambda i: (i, 0)),
              pl.BlockSpec((RPB, HID), lambda i: (i, 0)),
              pl.BlockSpec((1, HID), lambda i: (0, 0)),
              pl.BlockSpec((HID, 16), lambda i: (0, 0)),
              pl.BlockSpec((1, 16), lambda i: (0, 0)),
              pl.BlockSpec((1, 16), lambda i: (0, 0))],
    out_specs=[pl.BlockSpec((RPB, 16), lambda i: (i, 0)),
               pl.BlockSpec((RPB, 16), lambda i: (i, 0)),
               pl.BlockSpec((RPB, 16), lambda i: (i, 0))],
    out_shape=[jax.ShapeDtypeStruct((NROWS, 16), f32),
               jax.ShapeDtypeStruct((NROWS, 16), f32),
               jax.ShapeDtypeStruct((NROWS, 16), f32)],
)

_combine = pl.pallas_call(
    _comb_body,
    out_shape=jax.ShapeDtypeStruct((NROWS, 16), f32),
)


def _p1_body(src_hbm, dst_hbm, as_hbm, ad_hbm, z16_hbm,
             p_hbm, dena_hbm, denb_hbm,
             sidx, didx, as_v, ad_v, p_v, den_sh,
             sem_idx, sem_in):
    c = lax.axis_index("c")
    s = lax.axis_index("s")
    wid = c * 16 + s
    pltpu.sync_copy(z16_hbm.at[pl.ds(s * RPW, RPW)],
                    den_sh.at[pl.ds(s * RPW, RPW)])
    plsc.subcore_barrier()
    base_w = wid * (NBLK * B)

    def fire_idx(j, d4):
        base = base_w + j * B
        pltpu.async_copy(src_hbm.at[pl.ds(base, B)], sidx.at[d4],
                         sem_idx.at[d4])
        pltpu.async_copy(dst_hbm.at[pl.ds(base, B)], didx.at[d4],
                         sem_idx.at[d4])

    def wait_idx(j, d4):
        base = base_w + j * B
        pltpu.make_async_copy(src_hbm.at[pl.ds(base, B)], sidx.at[d4],
                              sem_idx.at[d4]).wait()
        pltpu.make_async_copy(dst_hbm.at[pl.ds(base, B)], didx.at[d4],
                              sem_idx.at[d4]).wait()

    def fire_gather(d4, d):
        pltpu.async_copy(as_hbm.at[sidx.at[d4]], as_v.at[d], sem_in.at[d])
        pltpu.async_copy(ad_hbm.at[didx.at[d4]], ad_v.at[d], sem_in.at[d])

    def wait_gather(d4, d):
        pltpu.make_async_copy(as_hbm.at[sidx.at[d4]], as_v.at[d],
                              sem_in.at[d]).wait()
        pltpu.make_async_copy(ad_hbm.at[didx.at[d4]], ad_v.at[d],
                              sem_in.at[d]).wait()

    def do_block(j, k):
        d4, d = k, k % 2
        base = base_w + j * B
        wait_gather(d4, d)

        @pl.when(j + 1 < NBLK)
        def _():
            wait_idx(j + 1, (k + 1) % 4)
            fire_gather((k + 1) % 4, (k + 1) % 2)

        @plsc.parallel_loop(0, B, unroll=4)
        def _(b):
            e = as_v[d, b, :] + ad_v[d, b, :]
            e = jnp.maximum(e, 0.2 * e)
            p_v[d, b, :] = jnp.exp(e)

        pltpu.sync_copy(p_v.at[d], p_hbm.at[pl.ds(base, B)])
        pltpu.sync_copy(p_v.at[d], den_sh.at[didx.at[d4]], add=True)

        @pl.when(j + 2 < NBLK)
        def _():
            fire_idx(j + 2, (k + 2) % 4)

    fire_idx(0, 0)
    fire_idx(1, 1)
    wait_idx(0, 0)
    fire_gather(0, 0)

    def quad(q, carry):
        j0 = 4 * q
        for k in range(4):
            do_block(j0 + k, k)
        return carry
    lax.fori_loop(0, NBLK // 4, quad, 0)
    do_block(NBLK - 1, 0)
    plsc.subcore_barrier()

    @pl.when(c == 0)
    def _():
        pltpu.sync_copy(den_sh.at[pl.ds(s * RPW, RPW)],
                        dena_hbm.at[pl.ds(s * RPW, RPW)])

    @pl.when(c == 1)
    def _():
        pltpu.sync_copy(den_sh.at[pl.ds(s * RPW, RPW)],
                        denb_hbm.at[pl.ds(s * RPW, RPW)])


_pass1 = pl.kernel(
    _p1_body,
    out_type=(jax.ShapeDtypeStruct((EP_PAD, 16), f32),
              jax.ShapeDtypeStruct((NROWS, 16), f32),
              jax.ShapeDtypeStruct((NROWS, 16), f32)),
    mesh=_mesh,
    scratch_types=[pltpu.VMEM((4, B), i32), pltpu.VMEM((4, B), i32),
                   pltpu.VMEM((2, B, 16), f32), pltpu.VMEM((2, B, 16), f32),
                   pltpu.VMEM((2, B, 16), f32),
                   pltpu.VMEM_SHARED((NROWS, 16), f32),
                   pltpu.SemaphoreType.DMA((4,)),
                   pltpu.SemaphoreType.DMA((2,))],
    compiler_params=pltpu.CompilerParams(use_tc_tiling_on_sc=False),
)


def _p2_body(src_hbm, dst_hbm, p_hbm, rd_hbm, h_hbm, z_hbm,
             alpha_hbm, oa_hbm, ob_hbm,
             sidx, didx, p_v, d0_v, pk_v, rows_v, out_sh,
             sem_idx, sem_in,
             wide):
    c = lax.axis_index("c")
    s = lax.axis_index("s")
    wid = c * 16 + s
    pltpu.sync_copy(z_hbm.at[pl.ds(s * RPW, RPW)],
                    out_sh.at[pl.ds(s * RPW, RPW)])
    plsc.subcore_barrier()
    base_w = wid * (NBLK * B)

    def fire_idx(j, d4):
        base = base_w + j * B
        pltpu.async_copy(src_hbm.at[pl.ds(base, B)], sidx.at[d4],
                         sem_idx.at[d4])
        pltpu.async_copy(dst_hbm.at[pl.ds(base, B)], didx.at[d4],
                         sem_idx.at[d4])

    def wait_idx(j, d4):
        base = base_w + j * B
        pltpu.make_async_copy(src_hbm.at[pl.ds(base, B)], sidx.at[d4],
                              sem_idx.at[d4]).wait()
        pltpu.make_async_copy(dst_hbm.at[pl.ds(base, B)], didx.at[d4],
                              sem_idx.at[d4]).wait()

    def fire_gather(j, d4, d):
        base = base_w + j * B
        pltpu.async_copy(h_hbm.at[sidx.at[d4]], rows_v.at[d], sem_in.at[d])
        pltpu.async_copy(rd_hbm.at[didx.at[d4]], d0_v.at[d], sem_in.at[d])
        pltpu.async_copy(p_hbm.at[pl.ds(base, B)], p_v.at[d], sem_in.at[d])

    def wait_gather(j, d4, d):
        base = base_w + j * B
        pltpu.make_async_copy(h_hbm.at[sidx.at[d4]], rows_v.at[d],
                              sem_in.at[d]).wait()
        pltpu.make_async_copy(rd_hbm.at[didx.at[d4]], d0_v.at[d],
                              sem_in.at[d]).wait()
        pltpu.make_async_copy(p_hbm.at[pl.ds(base, B)], p_v.at[d],
                              sem_in.at[d]).wait()

    lane = lax.iota(i32, 16)
    low8 = lane & 7

    def do_block(j, k):
        d4, d = k, k % 2
        base = base_w + j * B
        wait_gather(j, d4, d)

        @pl.when(j + 1 < NBLK)
        def _():
            wait_idx(j + 1, (k + 1) % 4)
            fire_gather(j + 1, (k + 1) % 4, (k + 1) % 2)

        @plsc.parallel_loop(0, B // 2, unroll=2)
        def _(i):
            b0 = 2 * i
            b1 = 2 * i + 1
            a0 = p_v[d, b0, :] * d0_v[d, b0, :]
            a1 = p_v[d, b1, :] * d0_v[d, b1, :]
            a1lo = a1.at[low8].get(mode="promise_in_bounds")
            pk_v[d, pl.ds(i * 16, 16)] = jnp.where(lane < 8, a0, a1lo)
            if wide:
                for hh in range(HEADS):
                    sl = pl.ds(hh * 16, 16)
                    rows_v[d, b0, sl] = rows_v[d, b0, sl] * a0[hh]
                    rows_v[d, b1, sl] = rows_v[d, b1, sl] * a1[hh]
            else:
                rows_v[d, b0, :] = rows_v[d, b0, :] * a0
                rows_v[d, b1, :] = rows_v[d, b1, :] * a1

        pltpu.sync_copy(pk_v.at[d], alpha_hbm.at[pl.ds(base * 8, B * 8)])
        pltpu.sync_copy(rows_v.at[d], out_sh.at[didx.at[d4]], add=True)

        @pl.when(j + 2 < NBLK)
        def _():
            fire_idx(j + 2, (k + 2) % 4)

    fire_idx(0, 0)
    fire_idx(1, 1)
    wait_idx(0, 0)
    fire_gather(0, 0, 0)

    def quad(q, carry):
        j0 = 4 * q
        for k in range(4):
            do_block(j0 + k, k)
        return carry
    lax.fori_loop(0, NBLK // 4, quad, 0)
    do_block(NBLK - 1, 0)
    plsc.subcore_barrier()

    @pl.when(c == 0)
    def _():
        pltpu.sync_copy(out_sh.at[pl.ds(s * RPW, RPW)],
                        oa_hbm.at[pl.ds(s * RPW, RPW)])

    @pl.when(c == 1)
    def _():
        pltpu.sync_copy(out_sh.at[pl.ds(s * RPW, RPW)],
                        ob_hbm.at[pl.ds(s * RPW, RPW)])


def _make_pass2(w):
    return pl.kernel(
        functools.partial(_p2_body, wide=(w == HID)),
        out_type=(jax.ShapeDtypeStruct((EP_PAD * 8,), f32),
                  jax.ShapeDtypeStruct((NROWS, w), f32),
                  jax.ShapeDtypeStruct((NROWS, w), f32)),
        mesh=_mesh,
        scratch_types=[pltpu.VMEM((4, B), i32), pltpu.VMEM((4, B), i32),
                       pltpu.VMEM((2, B, 16), f32), pltpu.VMEM((2, B, 16), f32),
                       pltpu.VMEM((2, B * 8), f32),
                       pltpu.VMEM((2, B, w), f32),
                       pltpu.VMEM_SHARED((NROWS, w), f32),
                       pltpu.SemaphoreType.DMA((4,)),
                       pltpu.SemaphoreType.DMA((2,))],
        compiler_params=pltpu.CompilerParams(use_tc_tiling_on_sc=False),
    )


_pass2_big = _make_pass2(HID)
_pass2_small = _make_pass2(16)


def kernel(x, edge_index, W1, as1, ad1, b1, W2, as2, ad2, b2, W3, as3, ad3, b3):
    loops = jnp.arange(N, dtype=i32)
    pad = PAD_IDX + jnp.arange(EP_PAD - EP, dtype=i32) % (NROWS - N)
    src = jnp.concatenate([edge_index[0].astype(i32), loops, pad])
    dst = jnp.concatenate([edge_index[1].astype(i32), loops, pad])
    xp = jnp.pad(x, ((0, NROWS - N), (0, 0)))
    z16 = jnp.zeros((NROWS, 16), f32)
    z128 = jnp.zeros((NROWS, HID), f32)

    h1, as1t, ad1t = _prep1(xp, W1, as1, ad1)
    p1, d1a, d1b = _pass1(src, dst, as1t, ad1t, z16)
    rd1 = 1.0 / (d1a + d1b + 1e-16)
    a1f, o1a, o1b = _pass2_big(src, dst, p1, rd1, h1, z128)

    h2, as2t, ad2t = _prep2(o1a, o1b, b1.reshape(1, HID), W2, as2, ad2)
    p2, d2a, d2b = _pass1(src, dst, as2t, ad2t, z16)
    rd2 = 1.0 / (d2a + d2b + 1e-16)
    a2f, o2a, o2b = _pass2_big(src, dst, p2, rd2, h2, z128)

    W3p = jnp.pad(W3, ((0, 0), (0, 14)))
    as3p = jnp.pad(as3, ((0, 0), (0, 14)))
    ad3p = jnp.pad(ad3, ((0, 0), (0, 14)))
    h3t, as3t, ad3t = _prep3(o2a, o2b, b2.reshape(1, HID), W3p, as3p, ad3p)
    p3, d3a, d3b = _pass1(src, dst, as3t, ad3t, z16)
    rd3 = 1.0 / (d3a + d3b + 1e-16)
    a3f, o3a, o3b = _pass2_small(src, dst, p3, rd3, h3t, z16)

    out_full = _combine(o3a, o3b, jnp.pad(b3, (0, 14)).reshape(1, 16))
    a1 = a1f[:EP * HEADS].reshape(EP, HEADS)
    a2 = a2f[:EP * HEADS].reshape(EP, HEADS)
    a3 = a3f[:EP * HEADS].reshape(EP, HEADS)[:, :1]
    return (out_full[:N, :2], a1, a2, a3)
